# Initial kernel scaffold; baseline (speedup 1.0000x reference)
import jax
import jax.numpy as jnp
from jax.experimental import pallas as pl


def kernel(indices, embedding):
    raise NotImplementedError("write your pallas kernel here")


# speedup vs baseline: 2.2932x; 2.2932x over previous
import functools

import jax
import jax.numpy as jnp
from jax import lax
from jax.experimental import pallas as pl
from jax.experimental.pallas import tpu as pltpu
from jax.experimental.pallas import tpu_sc as plsc

BATCH = 16384
SEQ = 200
VOCAB = 15
EMBED = 32
N = BATCH * SEQ

NC = 2
NS = 16
NW = NC * NS
PER_W = N // NW
CHUNK = 1024
ITERS = PER_W // CHUNK
GROUPS = CHUNK // 16

_mesh = plsc.VectorSubcoreMesh(core_axis_name="c", subcore_axis_name="s")


@functools.partial(
    pl.kernel,
    out_type=jax.ShapeDtypeStruct((N * EMBED,), jnp.float32),
    mesh=_mesh,
    scratch_types=[
        pltpu.VMEM((VOCAB * EMBED,), jnp.float32),
        pltpu.VMEM((CHUNK,), jnp.int32),
        pltpu.VMEM((CHUNK * EMBED,), jnp.float32),
    ],
    compiler_params=pltpu.CompilerParams(needs_layout_passes=False),
)
def _emb_lookup(idx_hbm, table_hbm, out_hbm, table_v, idx_v, out_v):
    wid = lax.axis_index("s") * NC + lax.axis_index("c")
    pltpu.sync_copy(table_hbm, table_v)
    lanes = lax.broadcasted_iota(jnp.int32, (16,), 0)

    def chunk_body(it, carry):
        base = pl.multiple_of(wid * PER_W + it * CHUNK, CHUNK)
        pltpu.sync_copy(idx_hbm.at[pl.ds(base, CHUNK)], idx_v)

        def group_body(j, carry2):
            idxv = idx_v[pl.ds(j * 16, 16)]
            tbase = idxv * EMBED
            obase = j * (16 * EMBED) + lanes * EMBED
            for d in range(EMBED):
                vals = plsc.load_gather(table_v, [tbase + d])
                plsc.store_scatter(out_v, [obase + d], vals)
            return carry2

        lax.fori_loop(0, GROUPS, group_body, 0)
        pltpu.sync_copy(
            out_v, out_hbm.at[pl.ds(base * EMBED, CHUNK * EMBED)]
        )
        return carry

    lax.fori_loop(0, ITERS, chunk_body, 0)


def kernel(indices, embedding):
    idx_flat = indices.reshape(N)
    table_flat = embedding.reshape(VOCAB * EMBED)
    out = _emb_lookup(idx_flat, table_flat)
    return out.reshape(BATCH, SEQ, EMBED)


# speedup vs baseline: 2.8558x; 1.2453x over previous
import functools

import jax
import jax.numpy as jnp
from jax import lax
from jax.experimental import pallas as pl
from jax.experimental.pallas import tpu as pltpu
from jax.experimental.pallas import tpu_sc as plsc

BATCH = 16384
SEQ = 200
VOCAB = 15
EMBED = 32
N = BATCH * SEQ

NC = 2
NS = 16
NW = NC * NS
PER_W = N // NW
CHUNK = 1024
ITERS = PER_W // CHUNK
GROUPS = CHUNK // 16

_mesh = plsc.VectorSubcoreMesh(core_axis_name="c", subcore_axis_name="s")


@functools.partial(
    pl.kernel,
    out_type=jax.ShapeDtypeStruct((N * EMBED,), jnp.float32),
    mesh=_mesh,
    scratch_types=[
        pltpu.VMEM((VOCAB * EMBED,), jnp.float32),
        pltpu.VMEM((CHUNK,), jnp.int32),
        pltpu.VMEM((CHUNK * EMBED,), jnp.float32),
    ],
    compiler_params=pltpu.CompilerParams(needs_layout_passes=False),
)
def _emb_lookup(idx_hbm, table_hbm, out_hbm, table_v, idx_v, out_v):
    wid = lax.axis_index("s") * NC + lax.axis_index("c")
    pltpu.sync_copy(table_hbm, table_v)
    lanes = lax.broadcasted_iota(jnp.int32, (16,), 0)

    def chunk_body(it, carry):
        base = pl.multiple_of(wid * PER_W + it * CHUNK, CHUNK)
        pltpu.sync_copy(idx_hbm.at[pl.ds(base, CHUNK)], idx_v)

        @plsc.parallel_loop(0, GROUPS, unroll=4)
        def group_body(j):
            idxv = idx_v[pl.ds(j * 16, 16)]
            tbase = idxv * EMBED
            obase = j * (16 * EMBED) + lanes * EMBED
            for d in range(EMBED):
                vals = plsc.load_gather(table_v, [tbase + d])
                plsc.store_scatter(out_v, [obase + d], vals)
        pltpu.sync_copy(
            out_v, out_hbm.at[pl.ds(base * EMBED, CHUNK * EMBED)]
        )
        return carry

    lax.fori_loop(0, ITERS, chunk_body, 0)


def kernel(indices, embedding):
    idx_flat = indices.reshape(N)
    table_flat = embedding.reshape(VOCAB * EMBED)
    out = _emb_lookup(idx_flat, table_flat)
    return out.reshape(BATCH, SEQ, EMBED)


# speedup vs baseline: 6.3713x; 2.2310x over previous
import functools

import jax
import jax.numpy as jnp
from jax import lax
from jax.experimental import pallas as pl
from jax.experimental.pallas import tpu as pltpu
from jax.experimental.pallas import tpu_sc as plsc

BATCH = 16384
SEQ = 200
VOCAB = 15
EMBED = 32
N = BATCH * SEQ

NC = 2
NS = 16
NW = NC * NS
PER_W = N // NW
CHUNK = 1024
ITERS = PER_W // CHUNK
GROUPS = CHUNK // 16

_mesh = plsc.VectorSubcoreMesh(core_axis_name="c", subcore_axis_name="s")


@functools.partial(
    pl.kernel,
    out_type=jax.ShapeDtypeStruct((N * EMBED,), jnp.float32),
    mesh=_mesh,
    scratch_types=[
        pltpu.VMEM((VOCAB * EMBED,), jnp.float32),
        pltpu.VMEM((CHUNK,), jnp.int32),
        pltpu.VMEM((CHUNK * EMBED,), jnp.float32),
    ],
    compiler_params=pltpu.CompilerParams(needs_layout_passes=False),
)
def _emb_lookup(idx_hbm, table_hbm, out_hbm, table_v, idx_v, out_v):
    wid = lax.axis_index("s") * NC + lax.axis_index("c")
    pltpu.sync_copy(table_hbm, table_v)

    def chunk_body(it, carry):
        base = pl.multiple_of(wid * PER_W + it * CHUNK, CHUNK)
        pltpu.sync_copy(idx_hbm.at[pl.ds(base, CHUNK)], idx_v)

        @plsc.parallel_loop(0, GROUPS, unroll=2)
        def group_body(g):
            idxv = idx_v[pl.ds(g * 16, 16)]
            for k in range(16):
                row = idxv[k] * EMBED
                o = (g * 16 + k) * EMBED
                out_v[pl.ds(o, 16)] = table_v[pl.ds(row, 16)]
                out_v[pl.ds(o + 16, 16)] = table_v[pl.ds(row + 16, 16)]
        pltpu.sync_copy(
            out_v, out_hbm.at[pl.ds(base * EMBED, CHUNK * EMBED)]
        )
        return carry

    lax.fori_loop(0, ITERS, chunk_body, 0)


def kernel(indices, embedding):
    idx_flat = indices.reshape(N)
    table_flat = embedding.reshape(VOCAB * EMBED)
    out = _emb_lookup(idx_flat, table_flat)
    return out.reshape(BATCH, SEQ, EMBED)


# speedup vs baseline: 29.4577x; 4.6235x over previous
import functools

import jax
import jax.numpy as jnp
from jax import lax
from jax.experimental import pallas as pl
from jax.experimental.pallas import tpu as pltpu
from jax.experimental.pallas import tpu_sc as plsc

BATCH = 16384
SEQ = 200
VOCAB = 15
EMBED = 32
TW = 16

NC = 2
NS = 16
NW = NC * NS
NB_PER_W = BATCH // NW
NGROUPS = NB_PER_W // 16

_mesh = plsc.VectorSubcoreMesh(core_axis_name="c", subcore_axis_name="s")


@functools.partial(
    pl.kernel,
    out_type=jax.ShapeDtypeStruct((SEQ, EMBED, BATCH), jnp.float32),
    mesh=_mesh,
    scratch_types=[
        pltpu.VMEM((EMBED * TW,), jnp.float32),
        pltpu.VMEM((NB_PER_W,), jnp.int32),
        pltpu.VMEM((EMBED, NB_PER_W), jnp.float32),
    ],
    compiler_params=pltpu.CompilerParams(needs_layout_passes=False),
)
def _emb_lookup(idx_hbm, table_hbm, out_hbm, table_v, idx_v, out_v):
    wid = lax.axis_index("s") * NC + lax.axis_index("c")
    b0 = pl.multiple_of(wid * NB_PER_W, NB_PER_W)
    pltpu.sync_copy(table_hbm, table_v)

    def s_body(s, carry):
        src = pl.multiple_of(s * BATCH + b0, NB_PER_W)
        pltpu.sync_copy(idx_hbm.at[pl.ds(src, NB_PER_W)], idx_v)

        @plsc.parallel_loop(0, NGROUPS, unroll=2)
        def group_body(g):
            idxv = idx_v[pl.ds(g * 16, 16)]
            for d in range(EMBED):
                vals = plsc.load_gather(table_v, [idxv + d * TW])
                out_v[d, pl.ds(g * 16, 16)] = vals

        pltpu.sync_copy(out_v, out_hbm.at[s, :, pl.ds(b0, NB_PER_W)])
        return carry

    lax.fori_loop(0, SEQ, s_body, 0)


def kernel(indices, embedding):
    idx_sm = indices.T.reshape(SEQ * BATCH)
    table_t = (
        jnp.zeros((EMBED, TW), jnp.float32)
        .at[:, :VOCAB]
        .set(embedding.T)
        .reshape(EMBED * TW)
    )
    out_t = _emb_lookup(idx_sm, table_t)
    return jnp.transpose(out_t, (2, 0, 1))


# speedup vs baseline: 48.3561x; 1.6415x over previous
import functools

import jax
import jax.numpy as jnp
from jax import lax
from jax.experimental import pallas as pl
from jax.experimental.pallas import tpu as pltpu
from jax.experimental.pallas import tpu_sc as plsc

BATCH = 16384
SEQ = 200
VOCAB = 15
EMBED = 32
TW = 16

NC = 2
NS = 16
NW = NC * NS
NBW = 1024
NB_SLABS = BATCH // NBW
S_PER_W = SEQ // (NW // NB_SLABS)
NGROUPS = NBW // 16

_mesh = plsc.VectorSubcoreMesh(core_axis_name="c", subcore_axis_name="s")


@functools.partial(
    pl.kernel,
    out_type=jax.ShapeDtypeStruct((SEQ, EMBED, BATCH), jnp.float32),
    mesh=_mesh,
    scratch_types=[
        pltpu.VMEM((EMBED * TW,), jnp.float32),
        pltpu.VMEM((2, NBW), jnp.int32),
        pltpu.VMEM((2, EMBED, NBW), jnp.float32),
        pltpu.SemaphoreType.DMA((2,)),
        pltpu.SemaphoreType.DMA((2,)),
    ],
    compiler_params=pltpu.CompilerParams(needs_layout_passes=False),
)
def _emb_lookup(idx_hbm, table_hbm, out_hbm, table_v, idx_v, out_v, sem_i, sem_o):
    wid = lax.axis_index("s") * NC + lax.axis_index("c")
    b0 = pl.multiple_of((wid % NB_SLABS) * NBW, NBW)
    s0 = (wid // NB_SLABS) * S_PER_W
    pltpu.sync_copy(table_hbm, table_v)

    def idx_start(s, buf):
        src = pl.multiple_of((s0 + s) * BATCH + b0, NBW)
        pltpu.async_copy(idx_hbm.at[pl.ds(src, NBW)], idx_v.at[buf], sem_i.at[buf])

    idx_start(0, 0)
    idx_start(1, 1)

    def s_body(i, carry):
        buf = lax.rem(i, 2)
        pltpu.make_async_copy(
            idx_hbm.at[pl.ds(0, NBW)], idx_v.at[buf], sem_i.at[buf]
        ).wait()

        @pl.when(i >= 2)
        def _():
            pltpu.make_async_copy(
                out_v.at[buf], out_hbm.at[0, :, pl.ds(b0, NBW)], sem_o.at[buf]
            ).wait()

        @plsc.parallel_loop(0, NGROUPS, unroll=2)
        def group_body(g):
            idxv = idx_v[buf, pl.ds(g * 16, 16)]
            for d in range(EMBED):
                vals = plsc.load_gather(table_v, [idxv + d * TW])
                out_v[buf, d, pl.ds(g * 16, 16)] = vals

        pltpu.async_copy(
            out_v.at[buf], out_hbm.at[s0 + i, :, pl.ds(b0, NBW)], sem_o.at[buf]
        )

        @pl.when(i + 2 < S_PER_W)
        def _():
            idx_start(i + 2, buf)

        return carry

    lax.fori_loop(0, S_PER_W, s_body, 0)

    for buf in range(2):
        pltpu.make_async_copy(
            out_v.at[buf], out_hbm.at[0, :, pl.ds(b0, NBW)], sem_o.at[buf]
        ).wait()


def kernel(indices, embedding):
    idx_sm = indices.T.reshape(SEQ * BATCH)
    table_t = (
        jnp.zeros((EMBED, TW), jnp.float32)
        .at[:, :VOCAB]
        .set(embedding.T)
        .reshape(EMBED * TW)
    )
    out_t = _emb_lookup(idx_sm, table_t)
    return jnp.transpose(out_t, (2, 0, 1))


# speedup vs baseline: 53.3626x; 1.1035x over previous
import functools

import jax
import jax.numpy as jnp
from jax import lax
from jax.experimental import pallas as pl
from jax.experimental.pallas import tpu as pltpu
from jax.experimental.pallas import tpu_sc as plsc

BATCH = 16384
SEQ = 200
VOCAB = 15
EMBED = 32
TW = 16

NC = 2
NS = 16
NW = NC * NS
NBW = 1024
NB_SLABS = BATCH // NBW
S_PER_W = SEQ // (NW // NB_SLABS)
NGROUPS = NBW // 16

_mesh = plsc.VectorSubcoreMesh(core_axis_name="c", subcore_axis_name="s")


@functools.partial(
    pl.kernel,
    out_type=jax.ShapeDtypeStruct((SEQ, EMBED, BATCH), jnp.float32),
    mesh=_mesh,
    scratch_types=[
        pltpu.VMEM((EMBED * TW,), jnp.float32),
        pltpu.VMEM((2, 8, 128), jnp.int32),
        pltpu.VMEM((2, EMBED, NBW), jnp.float32),
        pltpu.SemaphoreType.DMA((2,)),
        pltpu.SemaphoreType.DMA((2,)),
    ],
    compiler_params=pltpu.CompilerParams(needs_layout_passes=False),
)
def _emb_lookup(idx_hbm, table_hbm, out_hbm, table_v, idx_v, out_v, sem_i, sem_o):
    wid = lax.axis_index("s") * NC + lax.axis_index("c")
    b0 = pl.multiple_of((wid % NB_SLABS) * NBW, NBW)
    bh0 = pl.multiple_of((wid % NB_SLABS) * (NBW // 128), NBW // 128)
    s0 = (wid // NB_SLABS) * S_PER_W
    pltpu.sync_copy(table_hbm, table_v)

    def idx_start(s, buf):
        sa = s0 + s
        pltpu.async_copy(
            idx_hbm.at[sa // 8, pl.ds(bh0, 8), lax.rem(sa, 8), :],
            idx_v.at[buf],
            sem_i.at[buf],
        )

    idx_start(0, 0)
    idx_start(1, 1)

    def s_body(i, carry):
        buf = lax.rem(i, 2)
        pltpu.make_async_copy(
            idx_hbm.at[0, pl.ds(0, 8), 0, :], idx_v.at[buf], sem_i.at[buf]
        ).wait()

        @pl.when(i >= 2)
        def _():
            pltpu.make_async_copy(
                out_v.at[buf], out_hbm.at[0, :, pl.ds(b0, NBW)], sem_o.at[buf]
            ).wait()

        @plsc.parallel_loop(0, NGROUPS, unroll=2)
        def group_body(g):
            idxv = idx_v[buf, g // 8, pl.ds(lax.rem(g, 8) * 16, 16)]
            for d in range(EMBED):
                vals = plsc.load_gather(table_v, [idxv + d * TW])
                out_v[buf, d, pl.ds(g * 16, 16)] = vals

        pltpu.async_copy(
            out_v.at[buf], out_hbm.at[s0 + i, :, pl.ds(b0, NBW)], sem_o.at[buf]
        )

        @pl.when(i + 2 < S_PER_W)
        def _():
            idx_start(i + 2, buf)

        return carry

    lax.fori_loop(0, S_PER_W, s_body, 0)

    for buf in range(2):
        pltpu.make_async_copy(
            out_v.at[buf], out_hbm.at[0, :, pl.ds(b0, NBW)], sem_o.at[buf]
        ).wait()


def kernel(indices, embedding):
    idx_sm = indices.reshape(BATCH // 128, 128, SEQ // 8, 8).transpose(
        2, 0, 3, 1
    )
    table_t = (
        jnp.zeros((EMBED, TW), jnp.float32)
        .at[:, :VOCAB]
        .set(embedding.T)
        .reshape(EMBED * TW)
    )
    out_t = _emb_lookup(idx_sm, table_t)
    return jnp.transpose(out_t, (2, 0, 1))
